# Initial kernel scaffold; baseline (speedup 1.0000x reference)
import jax
import jax.numpy as jnp
from jax.experimental import pallas as pl


def kernel(user, media, user_table, media_table):
    raise NotImplementedError("write your pallas kernel here")


# speedup vs baseline: 5.7377x; 5.7377x over previous
import jax
import jax.numpy as jnp
from jax import lax
from jax.experimental import pallas as pl
from jax.experimental.pallas import tpu as pltpu
from jax.experimental.pallas import tpu_sc as plsc

_NC = 2
_NS = 16
_NW = _NC * _NS
_E = 12
_EP = 16

_B = 4096
_LU = 20
_LM = 50
_CHUNK = 128
_UC = (_B * _LU) // (_NW * _CHUNK)
_MC = (_B * _LM) // (_NW * _CHUNK)


def _sc_gather(uidx, midx, ut16, mt16):
  mesh = plsc.VectorSubcoreMesh(core_axis_name="c", subcore_axis_name="s")
  mhalf = _MC // 2

  def body(uidx_hbm, midx_hbm, ut_hbm, mt_hbm, ue_hbm, me_hbm,
           uidx_v, midx_v, urows_v, mrows_v, usem, msem):
    wid = lax.axis_index("s") * _NC + lax.axis_index("c")
    pltpu.sync_copy(uidx_hbm.at[wid], uidx_v)
    pltpu.sync_copy(midx_hbm.at[wid], midx_v)

    def fire_u(j, carry):
      pltpu.async_copy(ut_hbm.at[uidx_v.at[j]], urows_v.at[j], usem)
      return carry

    def fire_m_half(p, j, carry):
      pltpu.async_copy(mt_hbm.at[midx_v.at[p * mhalf + j]], mrows_v.at[j], msem)
      return carry

    lax.fori_loop(0, _UC, fire_u, 0)
    lax.fori_loop(0, mhalf, lambda j, c: fire_m_half(0, j, c), 0)
    pltpu.make_async_copy(ue_hbm.at[wid], urows_v, usem).wait()
    pltpu.sync_copy(urows_v, ue_hbm.at[wid])
    pltpu.make_async_copy(me_hbm.at[wid, pl.ds(0, mhalf)], mrows_v, msem).wait()
    pltpu.sync_copy(mrows_v, me_hbm.at[wid, pl.ds(0, mhalf)])
    lax.fori_loop(0, mhalf, lambda j, c: fire_m_half(1, j, c), 0)
    pltpu.make_async_copy(me_hbm.at[wid, pl.ds(mhalf, mhalf)], mrows_v, msem).wait()
    pltpu.sync_copy(mrows_v, me_hbm.at[wid, pl.ds(mhalf, mhalf)])

  f = pl.kernel(
      body,
      out_type=[
          jax.ShapeDtypeStruct((_NW, _UC, _CHUNK, _EP), jnp.float32),
          jax.ShapeDtypeStruct((_NW, _MC, _CHUNK, _EP), jnp.float32),
      ],
      mesh=mesh,
      scratch_types=[
          pltpu.VMEM((_UC, _CHUNK), jnp.int32),
          pltpu.VMEM((_MC, _CHUNK), jnp.int32),
          pltpu.VMEM((_UC, _CHUNK, _EP), jnp.float32),
          pltpu.VMEM((_MC // 2, _CHUNK, _EP), jnp.float32),
          pltpu.SemaphoreType.DMA,
          pltpu.SemaphoreType.DMA,
      ],
      compiler_params=pltpu.CompilerParams(use_tc_tiling_on_sc=False),
  )
  return f(uidx, midx, ut16, mt16)


_BBLK = 64


def _tc_body(ue_ref, me_ref, out_ref):
  acc = jax.lax.dot_general(
      ue_ref[...], me_ref[...], (((2,), (2,)), ((0,), (0,))),
      preferred_element_type=jnp.float32)
  out_ref[...] = 1.0 / (1.0 + jnp.exp(-acc))


def _tc_compute(ue, me):
  return pl.pallas_call(
      _tc_body,
      grid=(_B // _BBLK,),
      in_specs=[
          pl.BlockSpec((_BBLK, _LU, _EP), lambda i: (i, 0, 0)),
          pl.BlockSpec((_BBLK, _LM, _EP), lambda i: (i, 0, 0)),
      ],
      out_specs=pl.BlockSpec((_BBLK, _LU, _LM), lambda i: (i, 0, 0)),
      out_shape=jax.ShapeDtypeStruct((_B, _LU, _LM), jnp.float32),
  )(ue, me)


def kernel(user, media, user_table, media_table):
  uidx = user.astype(jnp.int32).reshape(_NW, _UC, _CHUNK)
  midx = media.astype(jnp.int32).reshape(_NW, _MC, _CHUNK)
  ut16 = jnp.pad(user_table, ((0, 0), (0, _EP - _E)))
  mt16 = jnp.pad(media_table, ((0, 0), (0, _EP - _E)))
  ue4, me4 = _sc_gather(uidx, midx, ut16, mt16)
  ue = ue4.reshape(_B, _LU, _EP)
  me = me4.reshape(_B, _LM, _EP)
  return _tc_compute(ue, me)
